# baseline (speedup 1.0000x reference)
import jax
import jax.numpy as jnp
from jax.experimental import pallas as pl
from jax.experimental.pallas import tpu as pltpu


def _add_block(x_ref, t_ref, o_ref):
    o_ref[...] = x_ref[...] + t_ref[...]


def _tc_add(x, table, b_lo, b_hi, BS=2048):
    B, S, D = x.shape
    off = table.shape[0] - S
    nb = b_hi - b_lo
    return pl.pallas_call(
        _add_block,
        grid=(S // BS, nb),
        in_specs=[
            pl.BlockSpec((1, BS, D), lambda s, b: (b + b_lo, s, 0)),
            pl.BlockSpec((BS, D), lambda s, b: (s + off // BS, 0)),
        ],
        out_specs=pl.BlockSpec((1, BS, D), lambda s, b: (b, s, 0)),
        out_shape=jax.ShapeDtypeStruct((nb, S, D), x.dtype),
        compiler_params=pltpu.CompilerParams(
            dimension_semantics=("parallel", "parallel"),
        ),
    )(x, table)


def kernel(x, table):
    a = _tc_add(x, table, 0, 3)
    b = _tc_add(x, table, 3, 4)
    return jnp.concatenate([a, b], axis=0)


# speedup vs baseline: 2.0308x; 2.0308x over previous
import jax
import jax.numpy as jnp
from jax.experimental import pallas as pl
from jax.experimental.pallas import tpu as pltpu


def _add_block(x_ref, t_ref, o_ref):
    o_ref[...] = x_ref[...] + t_ref[...]


def _tc_add(x, table, b_lo, b_hi, BS=2048):
    B, S, D = x.shape
    off = table.shape[0] - S
    nb = b_hi - b_lo
    return pl.pallas_call(
        _add_block,
        grid=(S // BS, nb),
        in_specs=[
            pl.BlockSpec((1, BS, D), lambda s, b: (b + b_lo, s, 0)),
            pl.BlockSpec((BS, D), lambda s, b: (s + off // BS, 0)),
        ],
        out_specs=pl.BlockSpec((1, BS, D), lambda s, b: (b, s, 0)),
        out_shape=jax.ShapeDtypeStruct((nb, S, D), x.dtype),
        compiler_params=pltpu.CompilerParams(
            dimension_semantics=("parallel", "parallel"),
        ),
    )(x, table)


def kernel(x, table):
    B, S, D = x.shape
    off = table.shape[0] - S
    BS = 512
    return pl.pallas_call(
        _add_block,
        grid=(S // BS,),
        in_specs=[
            pl.BlockSpec((B, BS, D), lambda s: (0, s, 0)),
            pl.BlockSpec((BS, D), lambda s: (s + off // BS, 0)),
        ],
        out_specs=pl.BlockSpec((B, BS, D), lambda s: (0, s, 0)),
        out_shape=jax.ShapeDtypeStruct((B, S, D), x.dtype),
        compiler_params=pltpu.CompilerParams(
            dimension_semantics=("parallel",),
        ),
    )(x, table)
